# baseline (speedup 1.0000x reference)
import math
import functools
import jax
import jax.numpy as jnp
from jax.experimental import pallas as pl
from jax.experimental.pallas import tpu as pltpu

D_MODEL = 128
BATCH = 16384
N_BASE = 10000.0
_BLK = 512


def _tc_body(t_ref, dfull_ref, out_ref):
    tv = t_ref[...]
    arg = tv * dfull_ref[...]
    par = jax.lax.broadcasted_iota(jnp.int32, (_BLK, D_MODEL), 1) % 2
    out_ref[...] = jnp.where(par == 0, jnp.sin(arg), jnp.cos(arg)) * 0.2


@jax.jit
def _tc_call(tf, dfull):
    return pl.pallas_call(
        _tc_body,
        grid=(BATCH // _BLK,),
        in_specs=[
            pl.BlockSpec((_BLK, 1), lambda i: (i, 0)),
            pl.BlockSpec((1, D_MODEL), lambda i: (0, 0)),
        ],
        out_specs=pl.BlockSpec((_BLK, D_MODEL), lambda i: (i, 0)),
        out_shape=jax.ShapeDtypeStruct((BATCH, D_MODEL), jnp.float32),
    )(tf, dfull)


def kernel(pe, t):
    div = jnp.exp(
        jnp.arange(0, D_MODEL, 2, dtype=jnp.float32) * (-math.log(N_BASE) / D_MODEL)
    )
    dfull = jnp.repeat(div, 2).reshape(1, D_MODEL)
    tf = t.astype(jnp.float32).reshape(BATCH, 1)
    return _tc_call(tf, dfull)
